# Initial kernel scaffold; baseline (speedup 1.0000x reference)
import jax
import jax.numpy as jnp
from jax.experimental import pallas as pl


def kernel(nodes, senders, receivers, edges, W0, b0, We0, W1, b1, We1):
    raise NotImplementedError("write your pallas kernel here")


# speedup vs baseline: 13.4756x; 13.4756x over previous
import functools
import jax
import jax.numpy as jnp
from jax import lax
from jax.experimental import pallas as pl
from jax.experimental.pallas import tpu as pltpu
from jax.experimental.pallas import tpu_sc as plsc

_N = 10000
_E = 320000
_D = 128
_DE = 16
_WE = 32

_CH = 128
_NACC = 10240
_DUMMY = _N
_RB = 1024
_NRB = _NACC // _RB

_PER_E = (2 * _E) // 32
_NCH_E = -(-_PER_E // _CH) + (-(-_PER_E // _CH)) % 2
_PERP_E = _NCH_E * _CH

_PER_G = (2 * _E) // 16
_NCH_G = -(-_PER_G // _CH) + (-(-_PER_G // _CH)) % 2
_PERP_G = _NCH_G * _CH

_ROWS_G = 16 * _NCH_G
_ROWS_E = 32 * _NCH_E
_OFF_GG = 0
_OFF_GS = _ROWS_G
_OFF_EG = 2 * _ROWS_G
_OFF_ES = 2 * _ROWS_G + _ROWS_E
_ROWS_ALL = 2 * _ROWS_G + 2 * _ROWS_E


def _fetch_idx(bidx, rowid_v, dst, base, nblk, gsem):
    for t in range(nblk):
        for q in range(8):
            lanes = base + t * _CH + q * 16 + lax.iota(jnp.int32, 16)
            rowid_v[t, pl.ds(q * 16, 16)] = jnp.minimum(lanes, _ROWS_ALL - 1)
    for t in range(nblk):
        pltpu.async_copy(bidx.at[rowid_v.at[t]],
                         dst.at[pl.ds(t * _CH, _CH)], gsem).wait()


def _writeback(acc, out_core, rowid_v, buf, sid):
    for z in range(5):
        row = (sid * 5 + z) * 128
        for q in range(8):
            lanes = row + q * 16 + lax.iota(jnp.int32, 16)
            rowid_v[0, pl.ds(q * 16, 16)] = lanes
        pltpu.sync_copy(acc.at[pl.ds(row, 128)], buf.at[0])
        pltpu.sync_copy(buf.at[0], out_core.at[rowid_v.at[0]])


def _ring_body(table, gidx_v, sidx_v, buf, acc, gsem0, gsem1, nch):
    pltpu.async_copy(table.at[gidx_v.at[0]], buf.at[0], gsem0)

    def step(i, carry):
        c0 = 2 * i
        pltpu.make_async_copy(table.at[gidx_v.at[c0]], buf.at[0],
                              gsem0).wait()
        pltpu.async_copy(table.at[gidx_v.at[c0 + 1]], buf.at[1], gsem1)
        pltpu.sync_copy(buf.at[0], acc.at[sidx_v.at[c0]], add=True)
        pltpu.make_async_copy(table.at[gidx_v.at[c0 + 1]], buf.at[1],
                              gsem1).wait()
        nxt = jnp.minimum(c0 + 2, nch - 1)
        pltpu.async_copy(table.at[gidx_v.at[nxt]], buf.at[0], gsem0)
        pltpu.sync_copy(buf.at[1], acc.at[sidx_v.at[c0 + 1]], add=True)
        return carry

    lax.fori_loop(0, nch // 2, step, 0)
    pltpu.make_async_copy(table.at[gidx_v.at[0]], buf.at[0], gsem0).wait()


def _make_sc_edge():
    mesh = plsc.VectorSubcoreMesh(core_axis_name="c", subcore_axis_name="s")

    @functools.partial(
        pl.kernel,
        mesh=mesh,
        compiler_params=pltpu.CompilerParams(use_tc_tiling_on_sc=False),
        out_type=jax.ShapeDtypeStruct((2, _NACC, _WE), jnp.float32),
        scratch_types=[
            pltpu.VMEM((2, _CH), jnp.int32),
            pltpu.VMEM((2 * _CH, _CH), jnp.int32),
            pltpu.VMEM((2 * _CH, _CH), jnp.int32),
            pltpu.VMEM((2, _CH, _WE), jnp.float32),
            pltpu.VMEM_SHARED((_NACC, _WE), jnp.float32),
            pltpu.SemaphoreType.DMA,
            pltpu.SemaphoreType.DMA,
        ],
    )
    def k(table, bidx, zeros_h, out, rowid_v, gidx_v, sidx_v, buf, acc,
          gsem0, gsem1):
        cid = lax.axis_index("c")
        sid = lax.axis_index("s")
        wid = cid * 16 + sid
        _fetch_idx(bidx, rowid_v, gidx_v, _OFF_EG + wid * _NCH_E, 2, gsem0)
        _fetch_idx(bidx, rowid_v, sidx_v, _OFF_ES + wid * _NCH_E, 2, gsem0)
        for z in range(5):
            row = (sid * 5 + z) * 128
            pltpu.sync_copy(zeros_h, acc.at[pl.ds(row, 128)])
        plsc.subcore_barrier()
        _ring_body(table, gidx_v, sidx_v, buf, acc, gsem0, gsem1, _NCH_E)
        plsc.subcore_barrier()
        _writeback(acc, out.at[cid], rowid_v, buf, sid)

    return k


def _make_sc_node():
    mesh = plsc.VectorSubcoreMesh(core_axis_name="c", subcore_axis_name="s")

    @functools.partial(
        pl.kernel,
        mesh=mesh,
        compiler_params=pltpu.CompilerParams(use_tc_tiling_on_sc=False),
        out_type=jax.ShapeDtypeStruct((4, _NACC, 32), jnp.float32),
        scratch_types=[
            pltpu.VMEM((3, _CH), jnp.int32),
            pltpu.VMEM((3 * _CH, _CH), jnp.int32),
            pltpu.VMEM((3 * _CH, _CH), jnp.int32),
            pltpu.VMEM((2, _CH, 32), jnp.float32),
            pltpu.VMEM_SHARED((_NACC, 32), jnp.float32),
            pltpu.SemaphoreType.DMA,
            pltpu.SemaphoreType.DMA,
        ],
    )
    def k(table, bidx, zeros_h, out, rowid_v, gidx_v, sidx_v, buf, acc,
          gsem0, gsem1):
        cid = lax.axis_index("c")
        sid = lax.axis_index("s")
        _fetch_idx(bidx, rowid_v, gidx_v, _OFF_GG + sid * _NCH_G, 3, gsem0)
        _fetch_idx(bidx, rowid_v, sidx_v, _OFF_GS + sid * _NCH_G, 3, gsem0)
        for p in range(2):
            q = cid * 2 + p
            for z in range(5):
                row = (sid * 5 + z) * 128
                pltpu.sync_copy(zeros_h, acc.at[pl.ds(row, 128)])
            plsc.subcore_barrier()
            _ring_body(table.at[q], gidx_v, sidx_v, buf, acc, gsem0, gsem1,
                       _NCH_G)
            plsc.subcore_barrier()
            _writeback(acc, out.at[q], rowid_v, buf, sid)

    return k


def _t0(acc_e, We0, b0, We1, b1):
    def body(acc_ref, we0_ref, b0_ref, we1_ref, b1_ref, et_ref, dinv_ref):
        acc = acc_ref[0] + acc_ref[1]
        deg = acc[:, 16:17]
        dinv = jnp.where(deg > 0, lax.rsqrt(jnp.maximum(deg, 1.0)), 0.0)
        eagg = acc[:, :16]
        et_ref[0] = jnp.dot(eagg, we0_ref[...],
                            preferred_element_type=jnp.float32) + b0_ref[...]
        et_ref[1] = jnp.dot(eagg, we1_ref[...],
                            preferred_element_type=jnp.float32) + b1_ref[...]
        dinv_ref[...] = jnp.broadcast_to(dinv, (_RB, _D))

    return pl.pallas_call(
        body,
        grid=(_NRB,),
        in_specs=[
            pl.BlockSpec((2, _RB, _WE), lambda i: (0, i, 0)),
            pl.BlockSpec((_DE, _D), lambda i: (0, 0)),
            pl.BlockSpec((1, _D), lambda i: (0, 0)),
            pl.BlockSpec((_DE, _D), lambda i: (0, 0)),
            pl.BlockSpec((1, _D), lambda i: (0, 0)),
        ],
        out_specs=[
            pl.BlockSpec((2, _RB, _D), lambda i: (0, i, 0)),
            pl.BlockSpec((_RB, _D), lambda i: (i, 0)),
        ],
        out_shape=[
            jax.ShapeDtypeStruct((2, _NACC, _D), jnp.float32),
            jax.ShapeDtypeStruct((_NACC, _D), jnp.float32),
        ],
    )(acc_e, We0, b0, We1, b1)


def _tg(x, W, dinvb):
    def body(x_ref, w_ref, dinv_ref, g_ref):
        g = jnp.dot(x_ref[...], w_ref[...],
                    preferred_element_type=jnp.float32) * dinv_ref[...]
        for q in range(4):
            g_ref[q] = g[:, 32 * q:32 * (q + 1)]

    return pl.pallas_call(
        body,
        grid=(_NRB,),
        in_specs=[
            pl.BlockSpec((_RB, _D), lambda i: (i, 0)),
            pl.BlockSpec((_D, _D), lambda i: (0, 0)),
            pl.BlockSpec((_RB, _D), lambda i: (i, 0)),
        ],
        out_specs=pl.BlockSpec((4, _RB, 32), lambda i: (0, i, 0)),
        out_shape=jax.ShapeDtypeStruct((4, _NACC, 32), jnp.float32),
    )(x, W, dinvb)


def _tc(x, agg, dinvb, et):
    def body(x_ref, agg_ref, dinv_ref, et_ref, out_ref):
        agg = jnp.concatenate([agg_ref[q] for q in range(4)], axis=-1)
        out_ref[...] = x_ref[...] + dinv_ref[...] * agg + et_ref[...]

    return pl.pallas_call(
        body,
        grid=(_NRB,),
        in_specs=[
            pl.BlockSpec((_RB, _D), lambda i: (i, 0)),
            pl.BlockSpec((4, _RB, 32), lambda i: (0, i, 0)),
            pl.BlockSpec((_RB, _D), lambda i: (i, 0)),
            pl.BlockSpec((_RB, _D), lambda i: (i, 0)),
        ],
        out_specs=pl.BlockSpec((_RB, _D), lambda i: (i, 0)),
        out_shape=jax.ShapeDtypeStruct((_NACC, _D), jnp.float32),
    )(x, agg, dinvb, et)


def _prep_idx(idx, padval, nworkers, per, perp):
    idx = idx.reshape(nworkers, per)
    pad = jnp.full((nworkers, perp - per), padval, jnp.int32)
    return jnp.concatenate([idx, pad], axis=1).reshape(-1, _CH)


def kernel(nodes, senders, receivers, edges, W0, b0, We0, W1, b1, We1):
    s2 = jnp.concatenate([senders, receivers])
    r2 = jnp.concatenate([receivers, senders])
    eidx = jnp.arange(_E, dtype=jnp.int32)
    eidx2 = jnp.concatenate([eidx, eidx])

    bidx = jnp.concatenate([
        _prep_idx(s2, 0, 16, _PER_G, _PERP_G),
        _prep_idx(r2, _DUMMY, 16, _PER_G, _PERP_G),
        _prep_idx(eidx2, 0, 32, _PER_E, _PERP_E),
        _prep_idx(r2, _DUMMY, 32, _PER_E, _PERP_E),
    ])

    ea = jnp.concatenate(
        [edges, jnp.ones((_E, 1), jnp.float32),
         jnp.zeros((_E, _WE - _DE - 1), jnp.float32)], axis=1)
    nodesp = jnp.pad(nodes, ((0, _NACC - _N), (0, 0)))
    zwe = jnp.zeros((128, _WE), jnp.float32)
    z32 = jnp.zeros((128, 32), jnp.float32)
    b0r = b0.reshape(1, _D)
    b1r = b1.reshape(1, _D)

    sc_e = _make_sc_edge()
    sc_g = _make_sc_node()

    acc_e = sc_e(ea, bidx, zwe)
    ets, dinvb = _t0(acc_e, We0, b0r, We1, b1r)
    Ws = jnp.stack([W0, W1])

    def layer(x, wet):
        W, et = wet
        g = _tg(x, W, dinvb)
        agg = sc_g(g, bidx, z32)
        return _tc(x, agg, dinvb, et), 0.0

    outp, _ = lax.scan(layer, nodesp, (Ws, ets))
    return outp[:_N]
